# baseline (speedup 1.0000x reference)
import jax
import jax.numpy as jnp
from jax import lax
from jax.experimental import pallas as pl
from jax.experimental.pallas import tpu as pltpu, tpu_sc as plsc

SIZE_VOCAB = 1000000
D = 64
B = 16384
L = 50

NC = 2
NS = 16
LANES = 16
NW = NC * NS
BC = B // NW
NCH = BC // 128
NWAVE = 2


def _rsqrt(x):
    i = lax.bitcast_convert_type(x, jnp.int32)
    y = lax.bitcast_convert_type(
        jnp.int32(0x5F3759DF) - lax.shift_right_arithmetic(i, 1), jnp.float32)
    for _ in range(3):
        y = y * (1.5 - 0.5 * x * y * y)
    return y


def _body(cen_hbm, ctx_hbm, win_hbm, wout_hbm, out_hbm,
          ridx_v, hidx_v, poff_v, in_v, wave_v, invin_v, res_v, sem):
    wid = lax.axis_index("s") * NC + lax.axis_index("c")
    base = wid * BC
    lanes = lax.iota(jnp.int32, LANES)

    def halve_indices():
        for j in range(NCH):
            for k in range(8):
                v = ridx_v[j, pl.ds(k * LANES, LANES)]
                hidx_v[j, pl.ds(k * LANES, LANES)] = lax.shift_right_logical(v, 1)
                poff_v[pl.ds((j * 8 + k) * LANES, LANES)] = lax.shift_left(v & 1, 6)

    for j in range(NCH):
        pltpu.sync_copy(cen_hbm.at[pl.ds(base + j * 128, 128)], ridx_v.at[j])
    halve_indices()
    for w in range(NCH // 2):
        for j in range(2):
            pltpu.async_copy(win_hbm.at[hidx_v.at[w * 2 + j]],
                             wave_v.at[pl.ds(j * 128, 128), :], sem)
        for j in range(2):
            pltpu.make_async_copy(win_hbm.at[hidx_v.at[w * 2 + j]],
                                  wave_v.at[pl.ds(j * 128, 128), :], sem).wait()

        def cgrp(g, _):
            rows = g * LANES + lanes
            gpos = w * 256 + g * LANES + lanes
            po = plsc.load_gather(poff_v, [gpos])
            acc = jnp.zeros((LANES,), jnp.float32)
            for d in range(D):
                col = (lanes + d) & (D - 1)
                v = plsc.load_gather(wave_v, [rows, col + po])
                plsc.store_scatter(in_v, [gpos, col], v)
                acc += v * v
            invin_v[pl.ds(w * 256 + g * LANES, LANES)] = _rsqrt(acc)
            return ()

        lax.fori_loop(0, 256 // LANES, cgrp, (), unroll=False)

    def l_body(l, _):
        for j in range(NCH):
            pltpu.sync_copy(ctx_hbm.at[l, pl.ds(base + j * 128, 128)],
                            ridx_v.at[j])
        halve_indices()

        for w in range(NWAVE):
            for j in range(2):
                pltpu.async_copy(wout_hbm.at[hidx_v.at[w * 2 + j]],
                                 wave_v.at[pl.ds(j * 128, 128), :], sem)
            for j in range(2):
                pltpu.make_async_copy(wout_hbm.at[hidx_v.at[w * 2 + j]],
                                      wave_v.at[pl.ds(j * 128, 128), :],
                                      sem).wait()

            def g_body(g, _):
                rows = g * LANES + lanes
                gpos = w * 256 + g * LANES + lanes
                po = plsc.load_gather(poff_v, [gpos])
                acc_d = jnp.zeros((LANES,), jnp.float32)
                acc_s = jnp.zeros((LANES,), jnp.float32)
                for d in range(D):
                    col = (lanes + d) & (D - 1)
                    o = plsc.load_gather(wave_v, [rows, col + po])
                    i = plsc.load_gather(in_v, [gpos, col])
                    acc_d += o * i
                    acc_s += o * o
                res = (acc_d * _rsqrt(acc_s)
                       * invin_v[pl.ds(w * 256 + g * LANES, LANES)])
                res_v[pl.ds(w * 256 + g * LANES, LANES)] = res
                return ()

            lax.fori_loop(0, 256 // LANES, g_body, (), unroll=False)

        pltpu.sync_copy(res_v, out_hbm.at[l, pl.ds(base, BC)])
        return ()

    lax.fori_loop(0, L, l_body, (), unroll=False)


@jax.jit
def kernel(center, context, emb_in_weight, emb_out_weight):
    win2 = emb_in_weight.reshape(SIZE_VOCAB // 2, 2 * D)
    wout2 = emb_out_weight.reshape(SIZE_VOCAB // 2, 2 * D)

    mesh = plsc.VectorSubcoreMesh(core_axis_name="c", subcore_axis_name="s")
    f = pl.kernel(
        _body,
        out_type=jax.ShapeDtypeStruct((L, B), jnp.float32),
        mesh=mesh,
        compiler_params=pltpu.CompilerParams(
            needs_layout_passes=False, use_tc_tiling_on_sc=True),
        scratch_types=[
            pltpu.VMEM((NCH, 128), jnp.int32),
            pltpu.VMEM((NCH, 128), jnp.int32),
            pltpu.VMEM((BC,), jnp.int32),
            pltpu.VMEM((BC, D), jnp.float32),
            pltpu.VMEM((256, 2 * D), jnp.float32),
            pltpu.VMEM((BC,), jnp.float32),
            pltpu.VMEM((BC,), jnp.float32),
            pltpu.SemaphoreType.DMA,
        ],
    )
    return f(center, context, win2, wout2)
